# Initial kernel scaffold; baseline (speedup 1.0000x reference)
import jax
import jax.numpy as jnp
from jax.experimental import pallas as pl


def kernel(x, mask_indices):
    raise NotImplementedError("write your pallas kernel here")


# speedup vs baseline: 4.1973x; 4.1973x over previous
import jax
import jax.numpy as jnp
from jax.experimental import pallas as pl
from jax.experimental.pallas import tpu as pltpu

_B, _D = 16384, 4096
_BLOCK_ROWS = 512
_IDX_PAD = 512


def _body(idx_ref, x_ref, o_ref, mask_ref):
    @pl.when(pl.program_id(0) == 0)
    def _():
        cols = jax.lax.broadcasted_iota(jnp.int32, (1, _D), 1)
        idx = idx_ref[...].reshape(_IDX_PAD, 1)
        hit = jnp.any(idx == cols, axis=0, keepdims=True)
        mask_ref[...] = jnp.where(hit, 0.0, 1.0)

    o_ref[...] = x_ref[...] * mask_ref[...]


def kernel(x, mask_indices):
    idx = mask_indices.astype(jnp.int32)
    n = idx.shape[0]
    idx = jnp.pad(idx, (0, _IDX_PAD - n), constant_values=_D)
    idx2d = idx.reshape(1, _IDX_PAD)

    grid = (_B // _BLOCK_ROWS,)
    return pl.pallas_call(
        _body,
        grid=grid,
        in_specs=[
            pl.BlockSpec((1, _IDX_PAD), lambda i: (0, 0)),
            pl.BlockSpec((_BLOCK_ROWS, _D), lambda i: (i, 0)),
        ],
        out_specs=pl.BlockSpec((_BLOCK_ROWS, _D), lambda i: (i, 0)),
        out_shape=jax.ShapeDtypeStruct((_B, _D), jnp.float32),
        scratch_shapes=[pltpu.VMEM((1, _D), jnp.float32)],
    )(idx2d, x)
